# baseline (speedup 1.0000x reference)
import functools

import jax
import jax.numpy as jnp
from jax import lax
from jax.experimental import pallas as pl
from jax.experimental.pallas import tpu as pltpu
from jax.experimental.pallas import tpu_sc as plsc

N = 2048
D = 768
F = 512
E = 8
K = 2
T = 128
G = 40
R = G * T
C = 512
NW = 32
AB = (N * K) // NW
TB = N // NW
NCH = 8
CHK = AB // NCH


def _route_body(x_ref, gw_ref, pos_ref, wts_ref, eft_ref, live_ref,
                xp_ref):
    x = x_ref[...]
    gw = gw_ref[...]
    xi = lax.bitcast_convert_type(x.astype(jnp.bfloat16), jnp.uint16)
    xw = xi.astype(jnp.int32)
    xp_ref[...] = xw[:, :D // 2] | (xw[:, D // 2:] << 16)
    logits = lax.dot_general(gw, x, (((1,), (1,)), ((), ())),
                             preferred_element_type=jnp.float32)
    m = jnp.max(logits, axis=0, keepdims=True)
    ex = jnp.exp(logits - m)
    p = ex / jnp.sum(ex, axis=0, keepdims=True)

    erow = lax.broadcasted_iota(jnp.int32, (E, N), 0)
    w0 = jnp.max(p, axis=0, keepdims=True)
    i0 = jnp.min(jnp.where(p == w0, erow, E), axis=0, keepdims=True)
    pm = jnp.where(erow == i0, -jnp.float32(jnp.inf), p)
    w1 = jnp.max(pm, axis=0, keepdims=True)
    i1 = jnp.min(jnp.where(pm == w1, erow, E), axis=0, keepdims=True)

    oh0 = (erow == i0).astype(jnp.float32)
    oh1 = (erow == i1).astype(jnp.float32)

    ri = lax.broadcasted_iota(jnp.int32, (C, C), 0)
    ci = lax.broadcasted_iota(jnp.int32, (C, C), 1)
    triu = (ri < ci).astype(jnp.float32)
    off = jnp.zeros((E, 1), jnp.float32)
    ranks = []
    for oh in (oh0, oh1):
        for c in range(N // C):
            mc = lax.slice(oh, (0, c * C), (E, (c + 1) * C))
            excl = lax.dot_general(mc, triu, (((1,), (0,)), ((), ())),
                                   preferred_element_type=jnp.float32)
            ranks.append(off + excl)
            off = off + jnp.sum(mc, axis=1, keepdims=True)
    counts = off
    aligned = jnp.floor((counts + (T - 1)) / T) * T

    er = lax.broadcasted_iota(jnp.int32, (E, E), 0)
    ec = lax.broadcasted_iota(jnp.int32, (E, E), 1)
    lt = (ec < er).astype(jnp.float32)
    start = lax.dot_general(lt, aligned, (((1,), (0,)), ((), ())),
                            preferred_element_type=jnp.float32)
    end = start + aligned

    tc = lax.broadcasted_iota(jnp.int32, (E, G), 1).astype(jnp.float32) * T
    eftp = jnp.sum((tc >= jnp.broadcast_to(end, (E, G))).astype(jnp.int32),
                   axis=0, keepdims=True)
    eft_ref[...] = jnp.minimum(eftp, E - 1)

    endu = start + counts
    au = jnp.sum((tc >= jnp.broadcast_to(endu, (E, G))).astype(jnp.int32),
                 axis=0, keepdims=True)
    live_ref[...] = ((au == eftp) & (eftp < E)).astype(jnp.int32)

    excl0 = jnp.concatenate(ranks[0:4], axis=1)
    excl1 = jnp.concatenate(ranks[4:8], axis=1)
    pos0 = jnp.sum((start + excl0) * oh0, axis=0, keepdims=True)
    pos1 = jnp.sum((start + excl1) * oh1, axis=0, keepdims=True)
    pos_ref[0:1, :] = pos0.astype(jnp.int32)
    pos_ref[1:2, :] = pos1.astype(jnp.int32)
    wts_ref[0:1, :] = w0
    wts_ref[1:2, :] = w1


_route = pl.pallas_call(
    _route_body,
    out_shape=(
        jax.ShapeDtypeStruct((K, N), jnp.int32),
        jax.ShapeDtypeStruct((K, N), jnp.float32),
        jax.ShapeDtypeStruct((1, G), jnp.int32),
        jax.ShapeDtypeStruct((1, G), jnp.int32),
        jax.ShapeDtypeStruct((N, D // 2), jnp.int32),
    ),
)


def _ffn_body(eft_ref, live_ref, d_ref, wr_ref, w1_ref, w3_ref, w2_ref,
              o_ref):
    del eft_ref
    i = pl.program_id(0)

    @pl.when(live_ref[i] == 1)
    def _():
        dw = d_ref[...]
        lo = lax.bitcast_convert_type(
            (dw & 0xFFFF).astype(jnp.uint16), jnp.bfloat16)
        hi = lax.bitcast_convert_type(
            lax.shift_right_logical(dw, 16).astype(jnp.uint16), jnp.bfloat16)
        d = jnp.concatenate([lo, hi], axis=1)
        a = lax.dot_general(d, w1_ref[0], (((1,), (1,)), ((), ())),
                            preferred_element_type=jnp.float32)
        b = lax.dot_general(d, w3_ref[0], (((1,), (1,)), ((), ())),
                            preferred_element_type=jnp.float32)
        h = a * lax.logistic(a) * b
        o = lax.dot_general(h, w2_ref[0], (((1,), (1,)), ((), ())),
                            preferred_element_type=jnp.float32)
        ob = lax.bitcast_convert_type(
            (o * wr_ref[...]).astype(jnp.bfloat16), jnp.uint16)
        ow = ob.astype(jnp.int32)
        o_ref[...] = ow[:, :D // 2] | (ow[:, D // 2:] << 16)


_ffn = pl.pallas_call(
    _ffn_body,
    grid_spec=pltpu.PrefetchScalarGridSpec(
        num_scalar_prefetch=2,
        grid=(G,),
        in_specs=[
            pl.BlockSpec((T, D // 2), lambda i, eft, lv: (i, 0)),
            pl.BlockSpec((T, 1), lambda i, eft, lv: (i, 0)),
            pl.BlockSpec((1, F, D), lambda i, eft, lv: (eft[i], 0, 0)),
            pl.BlockSpec((1, F, D), lambda i, eft, lv: (eft[i], 0, 0)),
            pl.BlockSpec((1, D, F), lambda i, eft, lv: (eft[i], 0, 0)),
        ],
        out_specs=pl.BlockSpec((T, D // 2), lambda i, eft, lv: (i, 0)),
    ),
    out_shape=jax.ShapeDtypeStruct((R, D // 2), jnp.int32),
)


@functools.cache
def _get_dispatch():
    mesh = plsc.VectorSubcoreMesh(core_axis_name="c", subcore_axis_name="s")

    @functools.partial(
        pl.kernel,
        out_type=(
            jax.ShapeDtypeStruct((R, D // 2), jnp.int32),
            jax.ShapeDtypeStruct((R,), jnp.float32),
        ),
        mesh=mesh,
        scratch_types=[
            pltpu.VMEM((NCH, CHK), jnp.int32),
            pltpu.VMEM((AB,), jnp.int32),
            pltpu.VMEM((AB,), jnp.float32),
            pltpu.VMEM((AB, D // 2), jnp.int32),
            [pltpu.SemaphoreType.DMA] * NCH,
            pltpu.SemaphoreType.DMA,
            pltpu.SemaphoreType.DMA,
        ],
    )
    def _dispatch(x_hbm, pos_hbm, wts_hbm, disp_hbm, wrow_hbm,
                  idx_v, idxf_v, w_v, rows_v, rsems, ssem, isem):
        wid = lax.axis_index("s") * 2 + lax.axis_index("c")
        a_base = wid * AB
        tok_base = (wid % (N // AB)) * AB
        smalls = [
            pltpu.async_copy(pos_hbm.at[pl.ds(a_base + c * CHK, CHK)],
                             idx_v.at[c], isem)
            for c in range(NCH)
        ]
        smalls.append(pltpu.async_copy(pos_hbm.at[pl.ds(a_base, AB)],
                                       idxf_v, isem))
        smalls.append(pltpu.async_copy(wts_hbm.at[pl.ds(a_base, AB)],
                                       w_v, isem))
        rcps = [
            pltpu.async_copy(x_hbm.at[pl.ds(tok_base + c * CHK, CHK)],
                             rows_v.at[pl.ds(c * CHK, CHK)], rsems[c])
            for c in range(NCH)
        ]
        for cp in smalls:
            cp.wait()
        wcp = pltpu.async_copy(w_v, wrow_hbm.at[idxf_v], ssem)
        scps = []
        for c in range(NCH):
            rcps[c].wait()
            scps.append(pltpu.async_copy(
                rows_v.at[pl.ds(c * CHK, CHK)],
                disp_hbm.at[idx_v.at[c]], ssem))
        for cp in scps:
            cp.wait()
        wcp.wait()

    return _dispatch


@functools.cache
def _get_combine():
    mesh = plsc.VectorSubcoreMesh(core_axis_name="c", subcore_axis_name="s")

    @functools.partial(
        pl.kernel,
        out_type=jax.ShapeDtypeStruct((N, D), jnp.float32),
        mesh=mesh,
        scratch_types=[
            pltpu.VMEM((TB,), jnp.int32),
            pltpu.VMEM((TB,), jnp.int32),
            pltpu.VMEM((TB, D // 2), jnp.int32),
            pltpu.VMEM((TB, D // 2), jnp.int32),
            pltpu.VMEM((TB, D), jnp.float32),
            pltpu.SemaphoreType.DMA,
        ],
    )
    def _combine(ybuf_hbm, pos_hbm, y_hbm,
                 idx0_v, idx1_v, r0_v, r1_v, y_v, sem):
        wid = lax.axis_index("s") * 2 + lax.axis_index("c")
        t_base = wid * TB
        pltpu.sync_copy(pos_hbm.at[pl.ds(t_base, TB)], idx0_v)
        pltpu.sync_copy(pos_hbm.at[pl.ds(N + t_base, TB)], idx1_v)
        cp0 = pltpu.async_copy(ybuf_hbm.at[idx0_v], r0_v, sem)
        cp1 = pltpu.async_copy(ybuf_hbm.at[idx1_v], r1_v, sem)
        cp0.wait()
        cp1.wait()

        def body(i, _):
            him = jnp.full((16,), -65536, jnp.int32)
            s16 = jnp.full((16,), 16, jnp.int32)
            for j in range(D // 32):
                sl = pl.ds(j * 16, 16)
                a = r0_v[i, sl]
                b = r1_v[i, sl]
                alo = lax.bitcast_convert_type(
                    lax.shift_left(a, s16), jnp.float32)
                blo = lax.bitcast_convert_type(
                    lax.shift_left(b, s16), jnp.float32)
                ahi = lax.bitcast_convert_type(a & him, jnp.float32)
                bhi = lax.bitcast_convert_type(b & him, jnp.float32)
                y_v[i, sl] = alo + blo
                y_v[i, pl.ds(D // 2 + j * 16, 16)] = ahi + bhi
            return 0

        lax.fori_loop(0, TB, body, 0)
        pltpu.sync_copy(y_v, y_hbm.at[pl.ds(t_base, TB)])

    return _combine


def kernel(x, gate_weight, w1, w2, w3):
    pos2, wts2, eft2, live2, xp = _route(x, gate_weight)
    pos = pos2.reshape(K * N)
    wts = wts2.reshape(K * N)
    disp, wrow = _get_dispatch()(xp, pos, wts)
    ybuf = _ffn(eft2.reshape(G), live2.reshape(G), disp,
                wrow.reshape(R, 1), w1, w3, w2)
    return _get_combine()(ybuf, pos)


# speedup vs baseline: 1.0245x; 1.0245x over previous
import functools

import jax
import jax.numpy as jnp
from jax import lax
from jax.experimental import pallas as pl
from jax.experimental.pallas import tpu as pltpu
from jax.experimental.pallas import tpu_sc as plsc

N = 2048
D = 768
F = 512
E = 8
K = 2
T = 128
G = 40
R = G * T
C = 512
NW = 32
AB = (N * K) // NW
TB = N // NW
NCH = 8
CHK = AB // NCH


def _route_body(x_ref, gw_ref, pos_ref, wts_ref, eft_ref, live_ref,
                xp_ref):
    x = x_ref[...]
    gw = gw_ref[...]
    xi = lax.bitcast_convert_type(x.astype(jnp.bfloat16), jnp.uint16)
    xw = xi.astype(jnp.int32)
    xp_ref[...] = xw[:, :D // 2] | (xw[:, D // 2:] << 16)
    logits = lax.dot_general(gw, x, (((1,), (1,)), ((), ())),
                             preferred_element_type=jnp.float32)
    m = jnp.max(logits, axis=0, keepdims=True)
    ex = jnp.exp(logits - m)
    p = ex / jnp.sum(ex, axis=0, keepdims=True)

    erow = lax.broadcasted_iota(jnp.int32, (E, N), 0)
    w0 = jnp.max(p, axis=0, keepdims=True)
    i0 = jnp.min(jnp.where(p == w0, erow, E), axis=0, keepdims=True)
    pm = jnp.where(erow == i0, -jnp.float32(jnp.inf), p)
    w1 = jnp.max(pm, axis=0, keepdims=True)
    i1 = jnp.min(jnp.where(pm == w1, erow, E), axis=0, keepdims=True)

    oh0 = (erow == i0).astype(jnp.float32)
    oh1 = (erow == i1).astype(jnp.float32)

    ri = lax.broadcasted_iota(jnp.int32, (C, C), 0)
    ci = lax.broadcasted_iota(jnp.int32, (C, C), 1)
    triu = (ri < ci).astype(jnp.float32)
    off = jnp.zeros((E, 1), jnp.float32)
    ranks = []
    for oh in (oh0, oh1):
        for c in range(N // C):
            mc = lax.slice(oh, (0, c * C), (E, (c + 1) * C))
            excl = lax.dot_general(mc, triu, (((1,), (0,)), ((), ())),
                                   preferred_element_type=jnp.float32)
            ranks.append(off + excl)
            off = off + jnp.sum(mc, axis=1, keepdims=True)
    counts = off
    aligned = jnp.floor((counts + (T - 1)) / T) * T

    er = lax.broadcasted_iota(jnp.int32, (E, E), 0)
    ec = lax.broadcasted_iota(jnp.int32, (E, E), 1)
    lt = (ec < er).astype(jnp.float32)
    start = lax.dot_general(lt, aligned, (((1,), (0,)), ((), ())),
                            preferred_element_type=jnp.float32)
    end = start + aligned

    tc = lax.broadcasted_iota(jnp.int32, (E, G), 1).astype(jnp.float32) * T
    eftp = jnp.sum((tc >= jnp.broadcast_to(end, (E, G))).astype(jnp.int32),
                   axis=0, keepdims=True)
    eft_ref[...] = jnp.minimum(eftp, E - 1)

    endu = start + counts
    au = jnp.sum((tc >= jnp.broadcast_to(endu, (E, G))).astype(jnp.int32),
                 axis=0, keepdims=True)
    live_ref[...] = ((au == eftp) & (eftp < E)).astype(jnp.int32)

    excl0 = jnp.concatenate(ranks[0:4], axis=1)
    excl1 = jnp.concatenate(ranks[4:8], axis=1)
    pos0 = jnp.sum((start + excl0) * oh0, axis=0, keepdims=True)
    pos1 = jnp.sum((start + excl1) * oh1, axis=0, keepdims=True)
    pos_ref[0:1, :] = pos0.astype(jnp.int32)
    pos_ref[1:2, :] = pos1.astype(jnp.int32)
    wts_ref[0:1, :] = w0
    wts_ref[1:2, :] = w1


_route = pl.pallas_call(
    _route_body,
    out_shape=(
        jax.ShapeDtypeStruct((K, N), jnp.int32),
        jax.ShapeDtypeStruct((K, N), jnp.float32),
        jax.ShapeDtypeStruct((1, G), jnp.int32),
        jax.ShapeDtypeStruct((1, G), jnp.int32),
        jax.ShapeDtypeStruct((N, D // 2), jnp.int32),
    ),
)


def _ffn_body(eft_ref, live_ref, d_ref, wr_ref, w1_ref, w3_ref, w2_ref,
              o_ref):
    del eft_ref
    i = pl.program_id(0)

    @pl.when(live_ref[i] == 1)
    def _():
        dw = d_ref[...]
        lo = lax.bitcast_convert_type(
            (dw & 0xFFFF).astype(jnp.uint16), jnp.bfloat16)
        hi = lax.bitcast_convert_type(
            lax.shift_right_logical(dw, 16).astype(jnp.uint16), jnp.bfloat16)
        d = jnp.concatenate([lo, hi], axis=1)
        a = lax.dot_general(d, w1_ref[0], (((1,), (1,)), ((), ())),
                            preferred_element_type=jnp.float32)
        b = lax.dot_general(d, w3_ref[0], (((1,), (1,)), ((), ())),
                            preferred_element_type=jnp.float32)
        h = a * lax.logistic(a) * b
        o = lax.dot_general(h, w2_ref[0], (((1,), (1,)), ((), ())),
                            preferred_element_type=jnp.float32)
        o_ref[...] = o * wr_ref[...]


_ffn = pl.pallas_call(
    _ffn_body,
    grid_spec=pltpu.PrefetchScalarGridSpec(
        num_scalar_prefetch=2,
        grid=(G,),
        in_specs=[
            pl.BlockSpec((T, D // 2), lambda i, eft, lv: (i, 0)),
            pl.BlockSpec((T, 1), lambda i, eft, lv: (i, 0)),
            pl.BlockSpec((1, F, D), lambda i, eft, lv: (eft[i], 0, 0)),
            pl.BlockSpec((1, F, D), lambda i, eft, lv: (eft[i], 0, 0)),
            pl.BlockSpec((1, D, F), lambda i, eft, lv: (eft[i], 0, 0)),
        ],
        out_specs=pl.BlockSpec((T, D), lambda i, eft, lv: (i, 0)),
    ),
    out_shape=jax.ShapeDtypeStruct((R, D), jnp.float32),
)


@functools.cache
def _get_dispatch():
    mesh = plsc.VectorSubcoreMesh(core_axis_name="c", subcore_axis_name="s")

    @functools.partial(
        pl.kernel,
        out_type=(
            jax.ShapeDtypeStruct((R, D // 2), jnp.int32),
            jax.ShapeDtypeStruct((R,), jnp.float32),
        ),
        mesh=mesh,
        scratch_types=[
            pltpu.VMEM((NCH, CHK), jnp.int32),
            pltpu.VMEM((AB,), jnp.int32),
            pltpu.VMEM((AB,), jnp.float32),
            pltpu.VMEM((AB, D // 2), jnp.int32),
            [pltpu.SemaphoreType.DMA] * NCH,
            pltpu.SemaphoreType.DMA,
            pltpu.SemaphoreType.DMA,
        ],
    )
    def _dispatch(x_hbm, pos_hbm, wts_hbm, disp_hbm, wrow_hbm,
                  idx_v, idxf_v, w_v, rows_v, rsems, ssem, isem):
        wid = lax.axis_index("s") * 2 + lax.axis_index("c")
        a_base = wid * AB
        tok_base = (wid % (N // AB)) * AB
        smalls = [
            pltpu.async_copy(pos_hbm.at[pl.ds(a_base + c * CHK, CHK)],
                             idx_v.at[c], isem)
            for c in range(NCH)
        ]
        smalls.append(pltpu.async_copy(pos_hbm.at[pl.ds(a_base, AB)],
                                       idxf_v, isem))
        smalls.append(pltpu.async_copy(wts_hbm.at[pl.ds(a_base, AB)],
                                       w_v, isem))
        rcps = [
            pltpu.async_copy(x_hbm.at[pl.ds(tok_base + c * CHK, CHK)],
                             rows_v.at[pl.ds(c * CHK, CHK)], rsems[c])
            for c in range(NCH)
        ]
        for cp in smalls:
            cp.wait()
        wcp = pltpu.async_copy(w_v, wrow_hbm.at[idxf_v], ssem)
        scps = []
        for c in range(NCH):
            rcps[c].wait()
            scps.append(pltpu.async_copy(
                rows_v.at[pl.ds(c * CHK, CHK)],
                disp_hbm.at[idx_v.at[c]], ssem))
        for cp in scps:
            cp.wait()
        wcp.wait()

    return _dispatch


@functools.cache
def _get_combine():
    mesh = plsc.VectorSubcoreMesh(core_axis_name="c", subcore_axis_name="s")

    @functools.partial(
        pl.kernel,
        out_type=jax.ShapeDtypeStruct((N, D), jnp.float32),
        mesh=mesh,
        scratch_types=[
            pltpu.VMEM((TB,), jnp.int32),
            pltpu.VMEM((TB,), jnp.int32),
            pltpu.VMEM((TB, D), jnp.float32),
            pltpu.VMEM((TB, D), jnp.float32),
            pltpu.SemaphoreType.DMA,
        ],
    )
    def _combine(ybuf_hbm, pos_hbm, y_hbm, idx0_v, idx1_v, r0_v, r1_v, sem):
        wid = lax.axis_index("s") * 2 + lax.axis_index("c")
        t_base = wid * TB
        pltpu.sync_copy(pos_hbm.at[pl.ds(t_base, TB)], idx0_v)
        pltpu.sync_copy(pos_hbm.at[pl.ds(N + t_base, TB)], idx1_v)
        cp0 = pltpu.async_copy(ybuf_hbm.at[idx0_v], r0_v, sem)
        cp1 = pltpu.async_copy(ybuf_hbm.at[idx1_v], r1_v, sem)
        cp0.wait()
        cp1.wait()

        def body(i, _):
            for j in range(D // 16):
                sl = pl.ds(j * 16, 16)
                r0_v[i, sl] = r0_v[i, sl] + r1_v[i, sl]
            return 0

        lax.fori_loop(0, TB, body, 0)
        pltpu.sync_copy(r0_v, y_hbm.at[pl.ds(t_base, TB)])

    return _combine


def kernel(x, gate_weight, w1, w2, w3):
    pos2, wts2, eft2, live2, xp = _route(x, gate_weight)
    pos = pos2.reshape(K * N)
    wts = wts2.reshape(K * N)
    disp, wrow = _get_dispatch()(xp, pos, wts)
    ybuf = _ffn(eft2.reshape(G), live2.reshape(G), disp,
                wrow.reshape(R, 1), w1, w3, w2)
    return _get_combine()(ybuf, pos)
